# Initial kernel scaffold; baseline (speedup 1.0000x reference)
import jax
import jax.numpy as jnp
from jax.experimental import pallas as pl


def kernel(rays_o, rays_d, z_vals, weights, u, num_importance):
    raise NotImplementedError("write your pallas kernel here")


# speedup vs baseline: 2.7619x; 2.7619x over previous
import functools

import jax
import jax.numpy as jnp
from jax import lax
from jax.experimental import pallas as pl
from jax.experimental.pallas import tpu as pltpu
from jax.experimental.pallas import tpu_sc as plsc

L = 16
NC, NS = 2, 16
NW = NC * NS
R = 16


def _vtake(x, idx):
    dnums = lax.GatherDimensionNumbers(
        offset_dims=(), collapsed_slice_dims=(0,), start_index_map=(0,)
    )
    return lax.gather(
        x, idx[:, None], dnums, (1,),
        mode=lax.GatherScatterMode.PROMISE_IN_BOUNDS,
    )


def _sampler(N, S, NI):
    SM = S - 1
    NCHUNK = S // L
    NU = NI // L
    RPW = N // NW
    NBLK = RPW // R

    mesh = plsc.VectorSubcoreMesh(
        core_axis_name="c", subcore_axis_name="s", num_cores=NC, num_subcores=NS
    )

    @functools.partial(
        pl.kernel,
        out_type=jax.ShapeDtypeStruct((N, NI), jnp.float32),
        mesh=mesh,
        compiler_params=pltpu.CompilerParams(needs_layout_passes=False),
        scratch_types=[
            pltpu.VMEM((R, S), jnp.float32),
            pltpu.VMEM((R, S), jnp.float32),
            pltpu.VMEM((R, NI), jnp.float32),
            pltpu.VMEM((R, NI), jnp.float32),
            pltpu.VMEM((S,), jnp.float32),
            pltpu.VMEM((S,), jnp.float32),
        ],
    )
    def body(z_hbm, w_hbm, u_hbm, out_hbm, zb, wb, ub, ob, cdf, zmid):
        wid = lax.axis_index("s") * NC + lax.axis_index("c")
        base0 = wid * RPW
        lane = lax.iota(jnp.int32, L)
        rot1 = (lane + 1) & (L - 1)

        def ray_body(r, _):
            carry = jnp.zeros((L,), jnp.float32)
            for c in range(NCHUNK):
                wv = wb[r, pl.ds(c * L, L)] + 1e-5
                if c == 0:
                    wv = jnp.where(lane >= 1, wv, 0.0)
                if c == NCHUNK - 1:
                    wv = jnp.where(lane <= L - 2, wv, 0.0)
                sc = plsc.cumsum(wv) + carry
                cdf[pl.ds(c * L, L)] = sc
                carry = _vtake(sc, jnp.full((L,), L - 1, jnp.int32))
            total = carry
            inv_total = 1.0 / total
            thresh = 1e-5 * total

            for c in range(NCHUNK):
                a = zb[r, pl.ds(c * L, L)]
                if c < NCHUNK - 1:
                    b = zb[r, pl.ds(c * L + 1, L)]
                else:
                    b = _vtake(a, rot1)
                zmid[pl.ds(c * L, L)] = 0.5 * (a + b)

            for k in range(NU):
                tv = ub[r, pl.ds(k * L, L)] * total
                lo = jnp.zeros((L,), jnp.int32)
                hi = jnp.full((L,), SM, jnp.int32)
                for _ in range(8):
                    mid = lax.shift_right_logical(lo + hi, 1)
                    cm = plsc.load_gather(cdf, [mid])
                    m = cm <= tv
                    lo = jnp.where(m, mid, lo)
                    hi = jnp.where(m, hi, mid)
                above = jnp.minimum(hi, SM - 1)
                g0 = plsc.load_gather(cdf, [lo])
                g1 = plsc.load_gather(cdf, [above])
                b0 = plsc.load_gather(zmid, [lo])
                b1 = plsc.load_gather(zmid, [above])
                num = tv - g0
                d = g1 - g0
                t = jnp.where(d < thresh, num * inv_total, num / d)
                ob[r, pl.ds(k * L, L)] = b0 + t * (b1 - b0)
            return 0

        def block_body(blk, _):
            base = base0 + blk * R
            pltpu.sync_copy(z_hbm.at[pl.ds(base, R)], zb)
            pltpu.sync_copy(w_hbm.at[pl.ds(base, R)], wb)
            pltpu.sync_copy(u_hbm.at[pl.ds(base, R)], ub)
            lax.fori_loop(0, R, ray_body, 0, unroll=False)
            pltpu.sync_copy(ob, out_hbm.at[pl.ds(base, R)])
            return 0

        lax.fori_loop(0, NBLK, block_body, 0, unroll=False)

    return body


def kernel(rays_o, rays_d, z_vals, weights, u, num_importance):
    del rays_o, rays_d, num_importance
    N, S = weights.shape
    NI = u.shape[1]
    return _sampler(N, S, NI)(z_vals, weights, u)
